# baseline (speedup 1.0000x reference)
import functools

import jax
import jax.numpy as jnp
from jax import lax
from jax.experimental import pallas as pl
from jax.experimental.pallas import tpu as pltpu
from jax.experimental.pallas import tpu_sc as plsc

_N_ACTIONS = 1000
_BATCH = 16384
_TC_BLK = 512
_TC_ROWS = 15872
_SC_ROWS = _BATCH - _TC_ROWS
_NW = 32
_RPW = _SC_ROWS // _NW
_NBLK = _BATCH // 8
_BPW = _RPW // 8
_CB = 2
_NCH = _BPW // _CB
_L = 16

_mesh = plsc.VectorSubcoreMesh(core_axis_name="c", subcore_axis_name="s")


def _tc_body(x_ref, idx_ref, out_ref):
    iota = lax.broadcasted_iota(jnp.int32, (_TC_BLK, _N_ACTIONS), 1)
    sel = jnp.where(iota == idx_ref[:][:, None], x_ref[:], 0.0)
    out_ref[:] = jnp.sum(sel, axis=1)


@functools.partial(
    pl.kernel,
    out_type=jax.ShapeDtypeStruct((_SC_ROWS,), jnp.float32),
    mesh=_mesh,
    scratch_types=[
        pltpu.VMEM((_RPW,), jnp.int32),
        pltpu.VMEM((_CB, 8, _N_ACTIONS), jnp.float32),
        pltpu.VMEM((_CB, 8, _N_ACTIONS), jnp.float32),
        pltpu.VMEM((_RPW,), jnp.float32),
        pltpu.SemaphoreType.DMA,
        pltpu.SemaphoreType.DMA,
    ],
    compiler_params=pltpu.CompilerParams(needs_layout_passes=False),
)
def _sc_kernel(mat_hbm, idx_hbm, out_hbm,
               idx_v, buf_a, buf_b, out_v, sem_a, sem_b):
    wid = lax.axis_index("s") * 2 + lax.axis_index("c")
    base = _TC_ROWS + wid * _RPW
    blk0 = _TC_ROWS // 8 + wid * _BPW

    pltpu.sync_copy(idx_hbm.at[pl.ds(base, _RPW)], idx_v)

    bufs = (buf_a, buf_b)
    sems = (sem_a, sem_b)
    copies = [None, None]
    rpc = _CB * 8

    def start(c):
        b = c % 2
        copies[b] = pltpu.async_copy(
            mat_hbm.at[pl.ds(blk0 + c * _CB, _CB)], bufs[b], sems[b])

    def extract(c):
        b = c % 2
        copies[b].wait()
        buf = bufs[b]
        for s in range(rpc // _L):
            off = c * rpc + s * _L
            cols = idx_v[pl.ds(off, _L)]
            local = s * _L + lax.iota(jnp.int32, _L)
            b16 = lax.shift_right_logical(local, 3)
            r16 = local & 7
            out_v[pl.ds(off, _L)] = plsc.load_gather(buf, [b16, r16, cols])

    start(0)
    for c in range(_NCH):
        if c + 1 < _NCH:
            start(c + 1)
        extract(c)

    pltpu.sync_copy(out_v, out_hbm.at[pl.ds(wid * _RPW, _RPW)])


_tc_call = pl.pallas_call(
    _tc_body,
    grid=(_TC_ROWS // _TC_BLK,),
    in_specs=[
        pl.BlockSpec((_TC_BLK, _N_ACTIONS), lambda i: (i, 0)),
        pl.BlockSpec((_TC_BLK,), lambda i: (i,)),
    ],
    out_specs=pl.BlockSpec((_TC_BLK,), lambda i: (i,)),
    out_shape=jax.ShapeDtypeStruct((_TC_ROWS,), jnp.float32),
)


def kernel(full_output, indices):
    idx = indices.astype(jnp.int32)
    mat = full_output.reshape(_NBLK, 8, _N_ACTIONS)
    out_sc = _sc_kernel(mat, idx)
    out_tc = _tc_call(full_output, idx)
    return jnp.concatenate([out_tc, out_sc])


# speedup vs baseline: 1.1534x; 1.1534x over previous
import functools

import jax
import jax.numpy as jnp
from jax import lax
from jax.experimental import pallas as pl
from jax.experimental.pallas import tpu as pltpu
from jax.experimental.pallas import tpu_sc as plsc

_N_ACTIONS = 1000
_BATCH = 16384
_NW = 32
_RPW = _BATCH // _NW
_NBLK = _BATCH // 8
_BPW = _NBLK // _NW
_CB = 4
_NCH = _BPW // _CB
_L = 16

_mesh = plsc.VectorSubcoreMesh(core_axis_name="c", subcore_axis_name="s")


@functools.partial(
    pl.kernel,
    out_type=jax.ShapeDtypeStruct((_BATCH,), jnp.float32),
    mesh=_mesh,
    scratch_types=[
        pltpu.VMEM((_RPW,), jnp.int32),
        pltpu.VMEM((_CB, 8, _N_ACTIONS), jnp.float32),
        pltpu.VMEM((_CB, 8, _N_ACTIONS), jnp.float32),
        pltpu.VMEM((_RPW,), jnp.float32),
        pltpu.SemaphoreType.DMA,
        pltpu.SemaphoreType.DMA,
    ],
    compiler_params=pltpu.CompilerParams(needs_layout_passes=False),
)
def _gather_kernel(mat_hbm, idx_hbm, out_hbm,
                   idx_v, buf_a, buf_b, out_v, sem_a, sem_b):
    wid = lax.axis_index("s") * 2 + lax.axis_index("c")
    base = wid * _RPW
    blk0 = wid * _BPW

    pltpu.sync_copy(idx_hbm.at[pl.ds(base, _RPW)], idx_v)

    bufs = (buf_a, buf_b)
    sems = (sem_a, sem_b)
    copies = [None, None]
    rpc = _CB * 8

    def start(c):
        b = c % 2
        copies[b] = pltpu.async_copy(
            mat_hbm.at[pl.ds(blk0 + c * _CB, _CB)], bufs[b], sems[b])

    def extract(c):
        b = c % 2
        copies[b].wait()
        buf = bufs[b]
        for s in range(rpc // _L):
            off = c * rpc + s * _L
            cols = idx_v[pl.ds(off, _L)]
            local = s * _L + lax.iota(jnp.int32, _L)
            b16 = lax.shift_right_logical(local, 3)
            r16 = local & 7
            out_v[pl.ds(off, _L)] = plsc.load_gather(buf, [b16, r16, cols])

    start(0)
    for c in range(_NCH):
        if c + 1 < _NCH:
            start(c + 1)
        extract(c)

    pltpu.sync_copy(out_v, out_hbm.at[pl.ds(base, _RPW)])


def kernel(full_output, indices):
    mat = full_output.reshape(_NBLK, 8, _N_ACTIONS)
    idx = indices.astype(jnp.int32)
    return _gather_kernel(mat, idx)
